# baseline (speedup 1.0000x reference)
import functools

import jax
import jax.numpy as jnp
from jax import lax
from jax.experimental import pallas as pl
from jax.experimental.pallas import tpu as pltpu
from jax.experimental.pallas import tpu_sc as plsc

N_NODES = 10000
N_EDGES = 320000
D = 128

NC = 2
NS = 16
NW = NC * NS
CHUNK = 128
BASE_CH = 78
LAST_CH = 82
EDGES_PER_W = BASE_CH * CHUNK
ROWS_PER_TILE = 640
LAST_ROWS = N_NODES - 15 * ROWS_PER_TILE

_mesh = plsc.VectorSubcoreMesh(core_axis_name="c", subcore_axis_name="s")

_DMA = pltpu.SemaphoreType.DMA


@functools.partial(
    pl.kernel,
    out_type=jax.ShapeDtypeStruct((NC, N_NODES, D), jnp.float32),
    mesh=_mesh,
    scratch_types=[
        pltpu.VMEM((1, CHUNK), jnp.int32),
        pltpu.VMEM((1, CHUNK), jnp.int32),
        pltpu.VMEM((1, CHUNK), jnp.int32),
        pltpu.VMEM((1, CHUNK), jnp.int32),
        pltpu.VMEM((CHUNK, D), jnp.float32),
        pltpu.VMEM((CHUNK, D), jnp.float32),
        pltpu.VMEM_SHARED((N_NODES, D), jnp.float32),
        _DMA, _DMA, _DMA, _DMA, _DMA, _DMA,
    ],
)
def _segsum_sc(edge_hbm, feat_hbm, zeros_hbm, out_hbm, *refs):
    SI = refs[0:2]
    DI = refs[2:4]
    R = refs[4:6]
    acc = refs[6]
    S = refs[7:9]
    Dm = refs[9:11]
    G = refs[11:13]

    c = lax.axis_index("c")
    s = lax.axis_index("s")
    w = s * NC + c
    base = w * EDGES_PER_W
    nch = jnp.where(w == NW - 1, LAST_CH, BASE_CH)

    def src_cp(q, m):
        off = pl.multiple_of(base + q * CHUNK, 8)
        return pltpu.async_copy(
            edge_hbm.at[pl.ds(0, 1), pl.ds(off, CHUNK)], SI[m], S[m])

    def dst_cp(q, m):
        off = pl.multiple_of(base + q * CHUNK, 8)
        return pltpu.async_copy(
            edge_hbm.at[pl.ds(1, 1), pl.ds(off, CHUNK)], DI[m], Dm[m])

    def swait(m):
        pltpu.make_async_copy(
            edge_hbm.at[pl.ds(0, 1), pl.ds(0, CHUNK)], SI[m], S[m]).wait()

    def dwait(m):
        pltpu.make_async_copy(
            edge_hbm.at[pl.ds(1, 1), pl.ds(0, CHUNK)], DI[m], Dm[m]).wait()

    def gather(m):
        return pltpu.async_copy(feat_hbm.at[SI[m].at[0]], R[m], G[m])

    def gwait(m):
        pltpu.make_async_copy(feat_hbm.at[SI[m].at[0]], R[m], G[m]).wait()

    def scatter(m):
        pltpu.sync_copy(R[m], acc.at[DI[m].at[0]], add=True)

    def phase(q, p, has_t, has_dst_pref, has_next, has_src_pref):
        o = 1 - p
        if has_dst_pref:
            dst_cp(q + 1, o)
        if has_next:
            swait(o)
            gather(o)
        gwait(p)
        if has_src_pref:
            src_cp(q + 2, p)
        dwait(p)
        scatter(p)

    src_cp(0, 0)
    src_cp(1, 1)
    dst_cp(0, 0)
    dst_cp(1, 1)
    swait(0)
    gather(0)
    pltpu.sync_copy(zeros_hbm, R[1])
    nz = jnp.where(s == NS - 1, LAST_ROWS // CHUNK, ROWS_PER_TILE // CHUNK)

    def zbody(z, carry):
        pltpu.sync_copy(
            R[1], acc.at[pl.ds(s * ROWS_PER_TILE + z * CHUNK, CHUNK)])
        return carry

    lax.fori_loop(0, nz, zbody, 0, unroll=False)

    ZREM = LAST_ROWS - (LAST_ROWS // CHUNK) * CHUNK

    @pl.when(s == NS - 1)
    def _():
        pltpu.sync_copy(
            R[1].at[pl.ds(0, ZREM)],
            acc.at[pl.ds((NS - 1) * ROWS_PER_TILE
                         + (LAST_ROWS // CHUNK) * CHUNK, ZREM)])

    plsc.subcore_barrier()

    phase(0, 0, False, False, True, True)
    phase(1, 1, True, True, True, True)

    def body(i, carry):
        q = 2 * i + 2
        phase(q, 0, True, True, True, True)
        phase(q + 1, 1, True, True, True, True)
        return carry

    lax.fori_loop(0, (nch - 4) // 2, body, 0, unroll=False)

    qT = nch - 2
    phase(qT, 0, True, True, True, False)
    phase(qT + 1, 1, True, False, False, False)
    plsc.subcore_barrier()

    @pl.when(s < NS - 1)
    def _():
        pltpu.sync_copy(acc.at[pl.ds(s * ROWS_PER_TILE, ROWS_PER_TILE)],
                        out_hbm.at[c, pl.ds(s * ROWS_PER_TILE, ROWS_PER_TILE)])

    @pl.when(s == NS - 1)
    def _():
        pltpu.sync_copy(acc.at[pl.ds((NS - 1) * ROWS_PER_TILE, LAST_ROWS)],
                        out_hbm.at[c, pl.ds((NS - 1) * ROWS_PER_TILE, LAST_ROWS)])


ROW_BLK = 1000


def _mm_body(p_ref, w_ref, b_ref, o_ref):
    h = p_ref[0] + p_ref[1]
    o_ref[...] = (
        lax.dot_general(h, w_ref[...], (((1,), (1,)), ((), ())),
                        preferred_element_type=jnp.float32)
        + b_ref[...]
    )


_mm_call = pl.pallas_call(
    _mm_body,
    grid=(N_NODES // ROW_BLK,),
    in_specs=[
        pl.BlockSpec((NC, ROW_BLK, D), lambda i: (0, i, 0)),
        pl.BlockSpec((D, D), lambda i: (0, 0)),
        pl.BlockSpec((1, D), lambda i: (0, 0)),
    ],
    out_specs=pl.BlockSpec((ROW_BLK, D), lambda i: (i, 0)),
    out_shape=jax.ShapeDtypeStruct((N_NODES, D), jnp.float32),
)


def kernel(feature, edge_index, W, b):
    zeros = jnp.zeros((CHUNK, D), jnp.float32)
    partials = _segsum_sc(edge_index, feature, zeros)
    return _mm_call(partials, W, b.reshape(1, D))


# speedup vs baseline: 1.0209x; 1.0209x over previous
import functools

import jax
import jax.numpy as jnp
from jax import lax
from jax.experimental import pallas as pl
from jax.experimental.pallas import tpu as pltpu
from jax.experimental.pallas import tpu_sc as plsc

N_NODES = 10000
N_EDGES = 320000
D = 128

NC = 2
NS = 16
NW = NC * NS
CHUNK = 128
BASE_CH = 78
LAST_CH = 82
EDGES_PER_W = BASE_CH * CHUNK
ROWS_PER_TILE = 640
LAST_ROWS = N_NODES - 15 * ROWS_PER_TILE

_mesh = plsc.VectorSubcoreMesh(core_axis_name="c", subcore_axis_name="s")

_DMA = pltpu.SemaphoreType.DMA


@functools.partial(
    pl.kernel,
    out_type=jax.ShapeDtypeStruct((NC, N_NODES, D), jnp.float32),
    mesh=_mesh,
    scratch_types=[
        pltpu.VMEM((1, CHUNK), jnp.int32),
        pltpu.VMEM((1, CHUNK), jnp.int32),
        pltpu.VMEM((1, CHUNK), jnp.int32),
        pltpu.VMEM((1, CHUNK), jnp.int32),
        pltpu.VMEM((CHUNK, D), jnp.float32),
        pltpu.VMEM((CHUNK, D), jnp.float32),
        pltpu.VMEM_SHARED((N_NODES, D), jnp.float32),
        _DMA, _DMA, _DMA, _DMA, _DMA, _DMA,
    ],
)
def _segsum_sc(edge_hbm, feat_hbm, zeros_hbm, out_hbm, *refs):
    SI = refs[0:2]
    DI = refs[2:4]
    R = refs[4:6]
    acc = refs[6]
    S = refs[7:9]
    Dm = refs[9:11]
    G = refs[11:13]

    c = lax.axis_index("c")
    s = lax.axis_index("s")
    w = s * NC + c
    base = w * EDGES_PER_W
    nch = jnp.where(w == NW - 1, LAST_CH, BASE_CH)

    def src_cp(q, m):
        off = pl.multiple_of(base + q * CHUNK, 8)
        return pltpu.async_copy(
            edge_hbm.at[pl.ds(0, 1), pl.ds(off, CHUNK)], SI[m], S[m])

    def dst_cp(q, m):
        off = pl.multiple_of(base + q * CHUNK, 8)
        return pltpu.async_copy(
            edge_hbm.at[pl.ds(1, 1), pl.ds(off, CHUNK)], DI[m], Dm[m])

    def swait(m):
        pltpu.make_async_copy(
            edge_hbm.at[pl.ds(0, 1), pl.ds(0, CHUNK)], SI[m], S[m]).wait()

    def dwait(m):
        pltpu.make_async_copy(
            edge_hbm.at[pl.ds(1, 1), pl.ds(0, CHUNK)], DI[m], Dm[m]).wait()

    def gather(m):
        return pltpu.async_copy(feat_hbm.at[SI[m].at[0]], R[m], G[m])

    def gwait(m):
        pltpu.make_async_copy(feat_hbm.at[SI[m].at[0]], R[m], G[m]).wait()

    def scatter(m):
        pltpu.sync_copy(R[m], acc.at[DI[m].at[0]], add=True)

    def phase(q, p, has_t, has_dst_pref, has_next, has_src_pref):
        o = 1 - p
        if has_dst_pref:
            dst_cp(q + 1, o)
        if has_next:
            swait(o)
            gather(o)
        gwait(p)
        if has_src_pref:
            src_cp(q + 2, p)
        dwait(p)
        scatter(p)

    src_cp(0, 0)
    src_cp(1, 1)
    dst_cp(0, 0)
    dst_cp(1, 1)
    swait(0)
    gather(0)
    pltpu.sync_copy(zeros_hbm, R[1])
    nz = jnp.where(s == NS - 1, LAST_ROWS // CHUNK, ROWS_PER_TILE // CHUNK)

    def zbody(z, carry):
        pltpu.sync_copy(
            R[1], acc.at[pl.ds(s * ROWS_PER_TILE + z * CHUNK, CHUNK)])
        return carry

    lax.fori_loop(0, nz, zbody, 0, unroll=False)

    ZREM = LAST_ROWS - (LAST_ROWS // CHUNK) * CHUNK

    @pl.when(s == NS - 1)
    def _():
        pltpu.sync_copy(
            R[1].at[pl.ds(0, ZREM)],
            acc.at[pl.ds((NS - 1) * ROWS_PER_TILE
                         + (LAST_ROWS // CHUNK) * CHUNK, ZREM)])

    plsc.subcore_barrier()

    phase(0, 0, False, False, True, True)
    phase(1, 1, True, True, True, True)

    def body(i, carry):
        q = 2 * i + 2
        phase(q, 0, True, True, True, True)
        phase(q + 1, 1, True, True, True, True)
        return carry

    lax.fori_loop(0, (nch - 4) // 2, body, 0, unroll=False)

    qT = nch - 2
    phase(qT, 0, True, True, True, False)
    phase(qT + 1, 1, True, False, False, False)
    plsc.subcore_barrier()

    @pl.when(s < NS - 1)
    def _():
        pltpu.sync_copy(acc.at[pl.ds(s * ROWS_PER_TILE, ROWS_PER_TILE)],
                        out_hbm.at[c, pl.ds(s * ROWS_PER_TILE, ROWS_PER_TILE)])

    @pl.when(s == NS - 1)
    def _():
        pltpu.sync_copy(acc.at[pl.ds((NS - 1) * ROWS_PER_TILE, LAST_ROWS)],
                        out_hbm.at[c, pl.ds((NS - 1) * ROWS_PER_TILE, LAST_ROWS)])


ROW_BLK = 2000


def _mm_body(p_ref, w_ref, b_ref, o_ref):
    h = p_ref[0] + p_ref[1]
    o_ref[...] = (
        lax.dot_general(h, w_ref[...], (((1,), (1,)), ((), ())),
                        preferred_element_type=jnp.float32)
        + b_ref[...]
    )


_mm_call = pl.pallas_call(
    _mm_body,
    grid=(N_NODES // ROW_BLK,),
    in_specs=[
        pl.BlockSpec((NC, ROW_BLK, D), lambda i: (0, i, 0)),
        pl.BlockSpec((D, D), lambda i: (0, 0)),
        pl.BlockSpec((1, D), lambda i: (0, 0)),
    ],
    out_specs=pl.BlockSpec((ROW_BLK, D), lambda i: (i, 0)),
    out_shape=jax.ShapeDtypeStruct((N_NODES, D), jnp.float32),
)


def kernel(feature, edge_index, W, b):
    zeros = jnp.zeros((CHUNK, D), jnp.float32)
    partials = _segsum_sc(edge_index, feature, zeros)
    return _mm_call(partials, W, b.reshape(1, D))
